# Initial kernel scaffold; baseline (speedup 1.0000x reference)
import jax
import jax.numpy as jnp
from jax.experimental import pallas as pl


def kernel(x, edge_index, batch, W1, b1, W2, b2, gate_w, gate_b, attn_w, attn_b, h1_w, h1_b, h2_w, h2_b):
    raise NotImplementedError("write your pallas kernel here")


# speedup vs baseline: 11.7573x; 11.7573x over previous
import functools

import jax
import jax.numpy as jnp
from jax import lax
from jax.experimental import pallas as pl
from jax.experimental.pallas import tpu as pltpu
from jax.experimental.pallas import tpu_sc as plsc

NC = 2
NS = 16
NW = NC * NS
CH = 128
DEG_W = 16
BN = 1024
GP = 128
G = 64

def _sc_mesh():
  return plsc.VectorSubcoreMesh(
      core_axis_name="c", subcore_axis_name="s", num_cores=NC, num_subcores=NS)


def _sc_degree(dst2d, npad, cpt):
  rpt = npad // NS

  @functools.partial(
      pl.kernel,
      mesh=_sc_mesh(),
      out_type=jax.ShapeDtypeStruct((NC, npad, DEG_W), jnp.float32),
      scratch_types=[
          pltpu.VMEM_SHARED((npad, DEG_W), jnp.float32),
          pltpu.VMEM((CH,), jnp.int32),
          pltpu.VMEM((CH, DEG_W), jnp.float32),
          pltpu.VMEM((rpt, DEG_W), jnp.float32),
      ],
  )
  def degk(dst_hbm, out_hbm, acc_sh, dstv, ones_v, zbuf):
    c = lax.axis_index("c")
    s = lax.axis_index("s")
    wid = c * NS + s

    def fill_zero(i, _):
      zbuf[i, :] = jnp.zeros((16,), jnp.float32)
      return 0

    lax.fori_loop(0, rpt, fill_zero, 0)
    pltpu.sync_copy(zbuf, acc_sh.at[pl.ds(s * rpt, rpt)])

    def fill_ones(i, _):
      ones_v[i, :] = jnp.ones((16,), jnp.float32)
      return 0

    lax.fori_loop(0, CH, fill_ones, 0)
    plsc.subcore_barrier()

    def chunk(j, _):
      pltpu.sync_copy(dst_hbm.at[wid * cpt + j], dstv)
      pltpu.sync_copy(ones_v, acc_sh.at[dstv], add=True)
      return 0

    lax.fori_loop(0, cpt, chunk, 0)
    plsc.subcore_barrier()
    pltpu.sync_copy(acc_sh.at[pl.ds(s * rpt, rpt)],
                    out_hbm.at[c, pl.ds(s * rpt, rpt)])

  return degk(dst2d)


def _sc_aggregate(hs, src2d, dst2d, npad, h, cpt):
  rpt = npad // NS

  @functools.partial(
      pl.kernel,
      mesh=_sc_mesh(),
      out_type=jax.ShapeDtypeStruct((NC, npad, h), jnp.float32),
      scratch_types=[
          pltpu.VMEM_SHARED((npad, h), jnp.float32),
          pltpu.VMEM((CH,), jnp.int32),
          pltpu.VMEM((CH,), jnp.int32),
          pltpu.VMEM((CH, h), jnp.float32),
          pltpu.SemaphoreType.DMA,
      ],
  )
  def aggk(hs_hbm, src_hbm, dst_hbm, out_hbm, acc_sh, srcv, dstv, rows, sem):
    c = lax.axis_index("c")
    s = lax.axis_index("s")
    wid = c * NS + s

    def fill_zero(i, _):
      for k in range(h // 16):
        rows[i, pl.ds(k * 16, 16)] = jnp.zeros((16,), jnp.float32)
      return 0

    lax.fori_loop(0, CH, fill_zero, 0)
    for k in range(rpt // CH):
      pltpu.sync_copy(rows, acc_sh.at[pl.ds(s * rpt + k * CH, CH)])
    plsc.subcore_barrier()

    def chunk(j, _):
      pltpu.sync_copy(src_hbm.at[wid * cpt + j], srcv)
      pltpu.sync_copy(dst_hbm.at[wid * cpt + j], dstv)
      pltpu.async_copy(hs_hbm.at[srcv], rows, sem).wait()
      pltpu.sync_copy(rows, acc_sh.at[dstv], add=True)
      return 0

    lax.fori_loop(0, cpt, chunk, 0)
    plsc.subcore_barrier()
    pltpu.sync_copy(acc_sh.at[pl.ds(s * rpt, rpt)],
                    out_hbm.at[c, pl.ds(s * rpt, rpt)])

  return aggk(hs, src2d, dst2d)


def _dinv_block(deg_ref):
  deg = deg_ref[0, :, 0] + deg_ref[1, :, 0] + 1.0
  return lax.rsqrt(deg)


def _tc_scale_matmul(xp, w, deg3, npad, f, h):
  def body(x_ref, w_ref, deg_ref, o_ref):
    dinv = _dinv_block(deg_ref)
    o_ref[...] = dinv[:, None] * jnp.dot(
        x_ref[...], w_ref[...], preferred_element_type=jnp.float32)

  return pl.pallas_call(
      body,
      grid=(npad // BN,),
      in_specs=[
          pl.BlockSpec((BN, f), lambda i: (i, 0)),
          pl.BlockSpec((f, h), lambda i: (0, 0)),
          pl.BlockSpec((NC, BN, DEG_W), lambda i: (0, i, 0)),
      ],
      out_specs=pl.BlockSpec((BN, h), lambda i: (i, 0)),
      out_shape=jax.ShapeDtypeStruct((npad, h), jnp.float32),
  )(xp, w, deg3)


def _tc_layer_next(acc, hs, deg3, w_next, b, npad, h):
  def body(a_ref, hs_ref, deg_ref, w_ref, b_ref, o_ref):
    dinv = _dinv_block(deg_ref)
    y = jnp.maximum(
        dinv[:, None] * (a_ref[0] + a_ref[1] + hs_ref[...]) + b_ref[...], 0.0)
    o_ref[...] = dinv[:, None] * jnp.dot(
        y, w_ref[...], preferred_element_type=jnp.float32)

  return pl.pallas_call(
      body,
      grid=(npad // BN,),
      in_specs=[
          pl.BlockSpec((NC, BN, h), lambda i: (0, i, 0)),
          pl.BlockSpec((BN, h), lambda i: (i, 0)),
          pl.BlockSpec((NC, BN, DEG_W), lambda i: (0, i, 0)),
          pl.BlockSpec((h, h), lambda i: (0, 0)),
          pl.BlockSpec((1, h), lambda i: (0, 0)),
      ],
      out_specs=pl.BlockSpec((BN, h), lambda i: (i, 0)),
      out_shape=jax.ShapeDtypeStruct((npad, h), jnp.float32),
  )(acc, hs, deg3, w_next, b)


def _tc_final_scores(acc, hs, deg3, b, gate_wp, gate_b2, batch2, npad, h):
  def body(a_ref, hs_ref, deg_ref, b_ref, gw_ref, gb_ref, bt_ref,
           y_ref, smax_ref):
    i = pl.program_id(0)
    dinv = _dinv_block(deg_ref)
    y = jnp.maximum(
        dinv[:, None] * (a_ref[0] + a_ref[1] + hs_ref[...]) + b_ref[...], 0.0)
    y_ref[...] = y
    s = jnp.dot(y, gw_ref[...], preferred_element_type=jnp.float32)[:, 0]
    s = s + gb_ref[0, 0]
    slots = lax.broadcasted_iota(jnp.int32, (BN, GP), 1)
    onehot = bt_ref[...] == slots
    masked = jnp.where(onehot, s[:, None], -jnp.inf)
    bmax = jnp.max(masked, axis=0)

    @pl.when(i == 0)
    def _():
      smax_ref[...] = jnp.full((8, GP), -jnp.inf, jnp.float32)

    smax_ref[...] = jnp.maximum(smax_ref[...], bmax[None, :])

  return pl.pallas_call(
      body,
      grid=(npad // BN,),
      in_specs=[
          pl.BlockSpec((NC, BN, h), lambda i: (0, i, 0)),
          pl.BlockSpec((BN, h), lambda i: (i, 0)),
          pl.BlockSpec((NC, BN, DEG_W), lambda i: (0, i, 0)),
          pl.BlockSpec((1, h), lambda i: (0, 0)),
          pl.BlockSpec((h, GP), lambda i: (0, 0)),
          pl.BlockSpec((1, GP), lambda i: (0, 0)),
          pl.BlockSpec((BN, 1), lambda i: (i, 0)),
      ],
      out_specs=[
          pl.BlockSpec((BN, h), lambda i: (i, 0)),
          pl.BlockSpec((8, GP), lambda i: (0, 0)),
      ],
      out_shape=[
          jax.ShapeDtypeStruct((npad, h), jnp.float32),
          jax.ShapeDtypeStruct((8, GP), jnp.float32),
      ],
  )(acc, hs, deg3, b, gate_wp, gate_b2, batch2)


def _tc_pool(y2, smax, batch2, gate_wp, gate_b2, attn_w, attn_b2, npad, h):
  def body(y_ref, sm_ref, bt_ref, gw_ref, gb_ref, aw_ref, ab_ref,
           num_ref, den_ref):
    i = pl.program_id(0)
    y = y_ref[...]
    s = jnp.dot(y, gw_ref[...], preferred_element_type=jnp.float32)[:, 0]
    s = s + gb_ref[0, 0]
    smax = sm_ref[0, :]
    smax = jnp.where(jnp.isfinite(smax), smax, 0.0)
    slots = lax.broadcasted_iota(jnp.int32, (BN, GP), 1)
    onehot = (bt_ref[...] == slots).astype(jnp.float32)
    srow = jnp.dot(onehot, smax, preferred_element_type=jnp.float32)
    alpha = jnp.exp(s - srow)
    ha = jnp.dot(y, aw_ref[...], preferred_element_type=jnp.float32)
    ha = ha + ab_ref[...]
    num = lax.dot_general(onehot, alpha[:, None] * ha,
                          (((0,), (0,)), ((), ())),
                          preferred_element_type=jnp.float32)
    den = jnp.sum(onehot * alpha[:, None], axis=0)

    @pl.when(i == 0)
    def _():
      num_ref[...] = jnp.zeros((GP, h), jnp.float32)
      den_ref[...] = jnp.zeros((8, GP), jnp.float32)

    num_ref[...] += num
    den_ref[...] += den[None, :]

  return pl.pallas_call(
      body,
      grid=(npad // BN,),
      in_specs=[
          pl.BlockSpec((BN, h), lambda i: (i, 0)),
          pl.BlockSpec((8, GP), lambda i: (0, 0)),
          pl.BlockSpec((BN, 1), lambda i: (i, 0)),
          pl.BlockSpec((h, GP), lambda i: (0, 0)),
          pl.BlockSpec((1, GP), lambda i: (0, 0)),
          pl.BlockSpec((h, h), lambda i: (0, 0)),
          pl.BlockSpec((1, h), lambda i: (0, 0)),
      ],
      out_specs=[
          pl.BlockSpec((GP, h), lambda i: (0, 0)),
          pl.BlockSpec((8, GP), lambda i: (0, 0)),
      ],
      out_shape=[
          jax.ShapeDtypeStruct((GP, h), jnp.float32),
          jax.ShapeDtypeStruct((8, GP), jnp.float32),
      ],
  )(y2, smax, batch2, gate_wp, gate_b2, attn_w, attn_b2)


def _tc_head(num, den, h1_wp, h1_b2, h2_wp, h2_b2, h):
  def body(num_ref, den_ref, w1_ref, b1_ref, w2_ref, b2_ref, o_ref):
    den = jnp.maximum(den_ref[0, :], 1e-12)
    pooled = num_ref[...] / den[:, None]
    z = jnp.maximum(
        jnp.dot(pooled, w1_ref[...], preferred_element_type=jnp.float32)
        + b1_ref[...], 0.0)
    o_ref[...] = jnp.dot(
        z, w2_ref[...], preferred_element_type=jnp.float32) + b2_ref[...]

  return pl.pallas_call(
      body,
      in_specs=[
          pl.BlockSpec((GP, h), lambda: (0, 0)),
          pl.BlockSpec((8, GP), lambda: (0, 0)),
          pl.BlockSpec((h, h), lambda: (0, 0)),
          pl.BlockSpec((1, h), lambda: (0, 0)),
          pl.BlockSpec((h, GP), lambda: (0, 0)),
          pl.BlockSpec((1, GP), lambda: (0, 0)),
      ],
      out_specs=pl.BlockSpec((GP, GP), lambda: (0, 0)),
      out_shape=jax.ShapeDtypeStruct((GP, GP), jnp.float32),
  )(num, den, h1_wp, h1_b2, h2_wp, h2_b2)


def kernel(x, edge_index, batch, W1, b1, W2, b2, gate_w, gate_b, attn_w,
           attn_b, h1_w, h1_b, h2_w, h2_b):
  n, f = x.shape
  h = W1.shape[1]
  h2 = h1_w.shape[1]
  e = edge_index.shape[1]

  npad = ((n + 1 + BN - 1) // BN) * BN
  cpt = -(-e // (NW * CH))
  epad = NW * cpt * CH

  src = edge_index[0]
  dst = edge_index[1]
  pad = jnp.full((epad - e,), n, jnp.int32)
  src2d = jnp.concatenate([src, pad]).reshape(NW * cpt, CH)
  dst2d = jnp.concatenate([dst, pad]).reshape(NW * cpt, CH)

  xp = jnp.zeros((npad, f), jnp.float32).at[:n].set(x.astype(jnp.float32))
  batch2 = jnp.concatenate(
      [batch.astype(jnp.int32),
       jnp.full((npad - n,), G, jnp.int32)]).reshape(npad, 1)

  gate_wp = jnp.zeros((h, GP), jnp.float32).at[:, :1].set(gate_w)
  gate_b2 = jnp.zeros((1, GP), jnp.float32) + gate_b[0]
  attn_b2 = attn_b.reshape(1, h)
  b1_2 = b1.reshape(1, h)
  b2_2 = b2.reshape(1, h)
  h1_wp = jnp.zeros((h, h), jnp.float32).at[:, :h2].set(h1_w)
  h1_b2 = jnp.zeros((1, h), jnp.float32).at[0, :h2].set(h1_b)
  h2_wp = jnp.zeros((h, GP), jnp.float32).at[:h2, :1].set(h2_w)
  h2_b2 = jnp.zeros((1, GP), jnp.float32) + h2_b[0]

  deg3 = _sc_degree(dst2d, npad, cpt)
  hs1 = _tc_scale_matmul(xp, W1, deg3, npad, f, h)
  acc1 = _sc_aggregate(hs1, src2d, dst2d, npad, h, cpt)
  hs2 = _tc_layer_next(acc1, hs1, deg3, W2, b1_2, npad, h)
  acc2 = _sc_aggregate(hs2, src2d, dst2d, npad, h, cpt)
  y2, smax = _tc_final_scores(acc2, hs2, deg3, b2_2, gate_wp, gate_b2, batch2,
                              npad, h)
  num, den = _tc_pool(y2, smax, batch2, gate_wp, gate_b2, attn_w, attn_b2,
                      npad, h)
  out = _tc_head(num, den, h1_wp, h1_b2, h2_wp, h2_b2, h)
  return out[:G, :1]
